# Initial kernel scaffold; baseline (speedup 1.0000x reference)
import jax
import jax.numpy as jnp
from jax.experimental import pallas as pl


def kernel(student_results, teacher_results, a_selected_indices, b_selected_indices):
    raise NotImplementedError("write your pallas kernel here")


# speedup vs baseline: 1.5577x; 1.5577x over previous
import functools

import jax
import jax.numpy as jnp
from jax import lax
from jax.experimental import pallas as pl
from jax.experimental.pallas import tpu as pltpu
from jax.experimental.pallas import tpu_sc as plsc

NC = 2
NS = 16
NW = NC * NS
LANES = 16
CHUNK = 32


def _build_gather(B, S, D, K):
    N = B * K
    rows_pw = N // NW
    jpw = rows_pw // CHUNK
    batches_per_worker = N // B // rows_pw

    mesh = plsc.VectorSubcoreMesh(core_axis_name="c", subcore_axis_name="s")

    @functools.partial(
        pl.kernel,
        out_type=[
            jax.ShapeDtypeStruct((N, D), jnp.float32),
            jax.ShapeDtypeStruct((N, D), jnp.float32),
            jax.ShapeDtypeStruct((N,), jnp.int32),
        ],
        mesh=mesh,
        scratch_types=[
            pltpu.VMEM((rows_pw,), jnp.int32),
            pltpu.VMEM((rows_pw,), jnp.int32),
            pltpu.VMEM((rows_pw,), jnp.int32),
            pltpu.VMEM((CHUNK, D), jnp.float32),
            pltpu.VMEM((CHUNK, D), jnp.float32),
            pltpu.SemaphoreType.DMA,
            pltpu.SemaphoreType.DMA,
            pltpu.SemaphoreType.DMA,
            pltpu.SemaphoreType.DMA,
        ],
    )
    def gather_kernel(student_hbm, teacher_hbm, a_idx_hbm, b_idx_hbm,
                      b_out, a_out, mask_out,
                      a_iv, b_iv, m_v, buf0, buf1, gs0, gs1, ws0, ws1):
        cid = lax.axis_index("c")
        sid = lax.axis_index("s")
        wid = sid * NC + cid
        row0 = wid * rows_pw
        pltpu.sync_copy(a_idx_hbm.at[pl.ds(row0, rows_pw)], a_iv)
        pltpu.sync_copy(b_idx_hbm.at[pl.ds(row0, rows_pw)], b_iv)
        batch_base = (wid // batches_per_worker) * S
        bb = jnp.full((LANES,), batch_base, dtype=jnp.int32)
        zero = jnp.zeros((LANES,), jnp.int32)
        one = jnp.ones((LANES,), jnp.int32)
        for t in range(rows_pw // LANES):
            sl = pl.ds(t * LANES, LANES)
            av = a_iv[sl]
            m_v[sl] = jnp.where(av > zero, one, zero)
            a_iv[sl] = av + bb
            b_iv[sl] = b_iv[sl] + bb
        pltpu.sync_copy(m_v, mask_out.at[pl.ds(row0, rows_pw)])

        bufs = (buf0, buf1)
        gsems = (gs0, gs1)
        wsems = (ws0, ws1)
        jobs = [(teacher_hbm, a_iv, a_out, j) for j in range(jpw)]
        jobs += [(student_hbm, b_iv, b_out, j) for j in range(jpw)]
        nj = len(jobs)

        def start_gather(i):
            table, iv, _, j = jobs[i]
            return pltpu.async_copy(
                table.at[iv.at[pl.ds(j * CHUNK, CHUNK)]], bufs[i % 2],
                gsems[i % 2])

        gh = [None] * nj
        wh = [None] * nj
        gh[0] = start_gather(0)
        for i in range(nj):
            if i + 1 < nj:
                if i - 1 >= 0:
                    wh[i - 1].wait()
                gh[i + 1] = start_gather(i + 1)
            gh[i].wait()
            _, _, out, j = jobs[i]
            wh[i] = pltpu.async_copy(
                bufs[i % 2], out.at[pl.ds(row0 + j * CHUNK, CHUNK)],
                wsems[i % 2])
        wh[nj - 2].wait()
        wh[nj - 1].wait()

    return gather_kernel


def kernel(student_results, teacher_results, a_selected_indices,
           b_selected_indices):
    B, S, D = student_results.shape
    K = a_selected_indices.shape[1]
    student_flat = student_results.reshape(B * S, D)
    teacher_flat = teacher_results.reshape(B * S, D)
    a_idx = a_selected_indices.astype(jnp.int32).reshape(B * K)
    b_idx = b_selected_indices.astype(jnp.int32).reshape(B * K)
    b_rows, a_rows, mask_i32 = _build_gather(B, S, D, K)(
        student_flat, teacher_flat, a_idx, b_idx)
    return (b_rows.reshape(B, K, D),
            a_rows.reshape(B, K, D),
            mask_i32.reshape(B, K).astype(jnp.bool_))


# speedup vs baseline: 1.5985x; 1.0262x over previous
import functools

import jax
import jax.numpy as jnp
from jax import lax
from jax.experimental import pallas as pl
from jax.experimental.pallas import tpu as pltpu
from jax.experimental.pallas import tpu_sc as plsc

NC = 2
NS = 16
NW = NC * NS
LANES = 16
CHUNK = 32
NBUF = 3


def _build_gather(B, S, D, K):
    N = B * K
    rows_pw = N // NW
    jpw = rows_pw // CHUNK
    batches_per_worker = N // B // rows_pw

    mesh = plsc.VectorSubcoreMesh(core_axis_name="c", subcore_axis_name="s")

    @functools.partial(
        pl.kernel,
        out_type=[
            jax.ShapeDtypeStruct((N, D), jnp.float32),
            jax.ShapeDtypeStruct((N, D), jnp.float32),
            jax.ShapeDtypeStruct((N,), jnp.int32),
        ],
        mesh=mesh,
        scratch_types=(
            [
                pltpu.VMEM((rows_pw,), jnp.int32),
                pltpu.VMEM((rows_pw,), jnp.int32),
                pltpu.VMEM((rows_pw,), jnp.int32),
            ]
            + [pltpu.VMEM((CHUNK, D), jnp.float32) for _ in range(NBUF)]
            + [pltpu.SemaphoreType.DMA for _ in range(2 * NBUF)]
        ),
    )
    def gather_kernel(student_hbm, teacher_hbm, a_idx_hbm, b_idx_hbm,
                      b_out, a_out, mask_out,
                      a_iv, b_iv, m_v, *bufs_and_sems):
        bufs = bufs_and_sems[:NBUF]
        gsems = bufs_and_sems[NBUF:2 * NBUF]
        wsems = bufs_and_sems[2 * NBUF:]
        cid = lax.axis_index("c")
        sid = lax.axis_index("s")
        wid = sid * NC + cid
        row0 = wid * rows_pw
        pltpu.sync_copy(a_idx_hbm.at[pl.ds(row0, rows_pw)], a_iv)
        pltpu.sync_copy(b_idx_hbm.at[pl.ds(row0, rows_pw)], b_iv)
        batch_base = (wid // batches_per_worker) * S
        bb = jnp.full((LANES,), batch_base, dtype=jnp.int32)
        zero = jnp.zeros((LANES,), jnp.int32)
        one = jnp.ones((LANES,), jnp.int32)
        for t in range(rows_pw // LANES):
            sl = pl.ds(t * LANES, LANES)
            av = a_iv[sl]
            m_v[sl] = jnp.where(av > zero, one, zero)
            a_iv[sl] = av + bb
            b_iv[sl] = b_iv[sl] + bb
        pltpu.sync_copy(m_v, mask_out.at[pl.ds(row0, rows_pw)])

        jobs = [(teacher_hbm, a_iv, a_out, j) for j in range(jpw)]
        jobs += [(student_hbm, b_iv, b_out, j) for j in range(jpw)]
        nj = len(jobs)

        def start_gather(i):
            table, iv, _, j = jobs[i]
            return pltpu.async_copy(
                table.at[iv.at[pl.ds(j * CHUNK, CHUNK)]], bufs[i % NBUF],
                gsems[i % NBUF])

        gh = [None] * nj
        wh = [None] * nj
        for i in range(min(NBUF, nj)):
            gh[i] = start_gather(i)
        for i in range(nj):
            gh[i].wait()
            _, _, out, j = jobs[i]
            wh[i] = pltpu.async_copy(
                bufs[i % NBUF], out.at[pl.ds(row0 + j * CHUNK, CHUNK)],
                wsems[i % NBUF])
            nxt = i + NBUF
            if nxt < nj:
                wh[i].wait()
                gh[nxt] = start_gather(nxt)
        for i in range(max(0, nj - NBUF), nj):
            if wh[i] is not None:
                wh[i].wait()

    return gather_kernel


def kernel(student_results, teacher_results, a_selected_indices,
           b_selected_indices):
    B, S, D = student_results.shape
    K = a_selected_indices.shape[1]
    student_flat = student_results.reshape(B * S, D)
    teacher_flat = teacher_results.reshape(B * S, D)
    a_idx = a_selected_indices.astype(jnp.int32).reshape(B * K)
    b_idx = b_selected_indices.astype(jnp.int32).reshape(B * K)
    b_rows, a_rows, mask_i32 = _build_gather(B, S, D, K)(
        student_flat, teacher_flat, a_idx, b_idx)
    return (b_rows.reshape(B, K, D),
            a_rows.reshape(B, K, D),
            mask_i32.reshape(B, K).astype(jnp.bool_))
